# baseline (speedup 1.0000x reference)
import jax
import jax.numpy as jnp
from jax import lax
from jax.experimental import pallas as pl
from jax.experimental.pallas import tpu as pltpu
from jax.experimental.pallas import tpu_sc as plsc

_ROWS = 2048
_WIDTH = 1024
_BATCH = 4
_NW = 32
_RPW = _ROWS // _NW
_CH = 16
_NCH = _RPW // _CH
_BP = 2
_NBP = _BATCH // _BP
_LANES = 16


def _sc_body(in_hbm, symv_hbm, table_hbm, out_hbm,
             idx_a, idx_b, emd_a, emd_b, sym_v, in_a, in_b,
             sem_ga, sem_gb, sem_ia, sem_ib, sem_oa, sem_ob):
    cid = lax.axis_index("c")
    sid = lax.axis_index("s")
    wid = cid * 16 + sid
    base = wid * _RPW

    pltpu.sync_copy(symv_hbm, sym_v)
    symv = sym_v[...]

    idx_refs = (idx_a, idx_b)
    emd_refs = (emd_a, emd_b)
    gsems = (sem_ga, sem_gb)
    bufs = (in_a, in_b)
    isems = (sem_ia, sem_ib)
    osems = (sem_oa, sem_ob)

    def gather_emd(c):
        p = c % 2
        rows = base + c * _CH + jnp.arange(_LANES, dtype=jnp.int32)
        idx_refs[p][...] = jnp.where(rows < symv, rows, jnp.int32(_ROWS - 1))
        return pltpu.async_copy(table_hbm.at[idx_refs[p]], emd_refs[p],
                                gsems[p])

    steps = [(c, bp) for c in range(_NCH) for bp in range(_NBP)]

    def in_copy(s):
        c, bp = steps[s]
        p = s % 2
        src = in_hbm.at[pl.ds(bp * _BP, _BP), pl.ds(base + c * _CH, _CH), :]
        return pltpu.async_copy(src, bufs[p], isems[p])

    def out_copy(s):
        c, bp = steps[s]
        p = s % 2
        dst = out_hbm.at[pl.ds(bp * _BP, _BP), pl.ds(base + c * _CH, _CH), :]
        return pltpu.async_copy(bufs[p], dst, osems[p])

    gh = [gather_emd(0), gather_emd(1)]
    ih = [in_copy(0), in_copy(1)]
    oh = [None, None]
    zero = jnp.zeros((_LANES,), jnp.float32)

    for s, (c, bp) in enumerate(steps):
        p = s % 2
        buf = bufs[p]
        emd = emd_refs[c % 2]
        if bp == 0 and gh[c % 2] is not None:
            gh[c % 2].wait()
            gh[c % 2] = None
        ih[p].wait()
        if oh[p] is not None:
            oh[p].wait()

        def add_blk(g, _, buf=buf, emd=emd, c=c):
            pr = lax.shift_right_logical(g, 3)
            bi = lax.shift_right_logical(pr, 4)
            row = lax.bitwise_and(pr, 15)
            col0 = lax.mul(lax.bitwise_and(g, 7), 128)
            is_sym = jnp.full((_LANES,), base + c * _CH + row,
                              jnp.int32) == symv
            for u in range(8):
                col = col0 + u * _LANES
                e = jnp.where(is_sym, zero, emd[row, pl.ds(col, _LANES)])
                plsc.addupdate(buf.at[bi, row, pl.ds(col, _LANES)], e)
            return 0

        lax.fori_loop(0, _BP * _CH * 8, add_blk, 0)

        oh[p] = out_copy(s)
        if s + 2 < len(steps):
            ih[p] = in_copy(s + 2)
        if bp == _NBP - 1 and c + 2 < _NCH:
            gh[c % 2] = gather_emd(c + 2)
    oh[0].wait()
    oh[1].wait()


def kernel(inputs, sym_index, table):
    symv = jnp.full((_LANES,), sym_index, jnp.int32)
    mesh = plsc.VectorSubcoreMesh(core_axis_name="c", subcore_axis_name="s")
    return pl.kernel(
        _sc_body,
        out_type=jax.ShapeDtypeStruct(inputs.shape, jnp.float32),
        mesh=mesh,
        scratch_types=[
            pltpu.VMEM((_CH,), jnp.int32),
            pltpu.VMEM((_CH,), jnp.int32),
            pltpu.VMEM((_CH, _WIDTH), jnp.float32),
            pltpu.VMEM((_CH, _WIDTH), jnp.float32),
            pltpu.VMEM((_LANES,), jnp.int32),
            pltpu.VMEM((_BP, _CH, _WIDTH), jnp.float32),
            pltpu.VMEM((_BP, _CH, _WIDTH), jnp.float32),
            pltpu.SemaphoreType.DMA,
            pltpu.SemaphoreType.DMA,
            pltpu.SemaphoreType.DMA,
            pltpu.SemaphoreType.DMA,
            pltpu.SemaphoreType.DMA,
            pltpu.SemaphoreType.DMA,
        ],
    )(inputs, symv, table)


# speedup vs baseline: 4.7703x; 4.7703x over previous
import jax
import jax.numpy as jnp
from jax.experimental import pallas as pl
from jax.experimental.pallas import tpu as pltpu

_ROWS = 2048
_WIDTH = 1024
_WBLK = 512


def _body(sym_ref, in_ref, tab_ref, last_ref, out_ref, emd_ref):
    b = pl.program_id(1)

    @pl.when(b == 0)
    def _compute_emd():
        sym = sym_ref[0]
        rows = jax.lax.broadcasted_iota(jnp.int32, (_ROWS, 1), 0)
        emd = jnp.where(rows < sym, tab_ref[...], last_ref[...])
        emd_ref[...] = jnp.where(rows == sym, jnp.float32(0.0), emd)

    out_ref[...] = in_ref[...] + emd_ref[...][None]


def kernel(inputs, sym_index, table):
    batch = inputs.shape[0]
    sym = jnp.asarray(sym_index, jnp.int32).reshape(1)
    last = table[-1:, :]
    grid = (_WIDTH // _WBLK, batch)
    return pl.pallas_call(
        _body,
        grid_spec=pltpu.PrefetchScalarGridSpec(
            num_scalar_prefetch=1,
            grid=grid,
            in_specs=[
                pl.BlockSpec((1, _ROWS, _WBLK), lambda w, b, sym: (b, 0, w)),
                pl.BlockSpec((_ROWS, _WBLK), lambda w, b, sym: (0, w)),
                pl.BlockSpec((1, _WBLK), lambda w, b, sym: (0, w)),
            ],
            out_specs=pl.BlockSpec((1, _ROWS, _WBLK), lambda w, b, sym: (b, 0, w)),
            scratch_shapes=[pltpu.VMEM((_ROWS, _WBLK), jnp.float32)],
        ),
        out_shape=jax.ShapeDtypeStruct(inputs.shape, inputs.dtype),
    )(sym, inputs, table, last)


# speedup vs baseline: 5.2113x; 1.0924x over previous
import jax
import jax.numpy as jnp
from jax.experimental import pallas as pl
from jax.experimental.pallas import tpu as pltpu

_ROWS = 2048
_WIDTH = 1024
_BATCH = 4
_BLK = 1024
_HALVES = _ROWS // _BLK


def _body(sym_ref, in_ref, tab_ref, last_ref, out_ref, emd_ref):
    r = pl.program_id(0)
    h = jax.lax.rem(r, _HALVES)

    @pl.when(r < _HALVES)
    def _compute_emd():
        sym = sym_ref[0]
        rows = h * _BLK + jax.lax.broadcasted_iota(jnp.int32, (_BLK, 1), 0)
        emd = jnp.where(rows < sym, tab_ref[...], last_ref[...])
        emd_ref[pl.ds(h * _BLK, _BLK), :] = jnp.where(
            rows == sym, jnp.float32(0.0), emd)

    out_ref[...] = in_ref[...] + emd_ref[pl.ds(h * _BLK, _BLK), :]


def _tab_idx(r, sym):
    h = jax.lax.rem(r, _HALVES)
    cap = jax.lax.div(jnp.maximum(sym[0] - 1, 0), _BLK)
    return (jnp.minimum(h, cap), 0)


def kernel(inputs, sym_index, table):
    sym = jnp.asarray(sym_index, jnp.int32).reshape(1)
    last = table[-1:, :]
    flat = inputs.reshape(_BATCH * _ROWS, _WIDTH)
    grid = (_BATCH * _ROWS // _BLK,)
    out = pl.pallas_call(
        _body,
        grid_spec=pltpu.PrefetchScalarGridSpec(
            num_scalar_prefetch=1,
            grid=grid,
            in_specs=[
                pl.BlockSpec((_BLK, _WIDTH), lambda r, sym: (r, 0)),
                pl.BlockSpec((_BLK, _WIDTH), _tab_idx),
                pl.BlockSpec((1, _WIDTH), lambda r, sym: (0, 0)),
            ],
            out_specs=pl.BlockSpec((_BLK, _WIDTH), lambda r, sym: (r, 0)),
            scratch_shapes=[pltpu.VMEM((_ROWS, _WIDTH), jnp.float32)],
        ),
        out_shape=jax.ShapeDtypeStruct((_BATCH * _ROWS, _WIDTH), inputs.dtype),
    )(sym, flat, table, last)
    return out.reshape(inputs.shape)


# speedup vs baseline: 5.2373x; 1.0050x over previous
import jax
import jax.numpy as jnp
from jax.experimental import pallas as pl
from jax.experimental.pallas import tpu as pltpu

_ROWS = 2048
_WIDTH = 1024
_BLK = 2048


def _body(sym_ref, in_ref, tab_ref, last_ref, out_ref, emd_ref):
    b = pl.program_id(1)

    @pl.when(b == 0)
    def _compute_emd():
        r = pl.program_id(0)
        sym = sym_ref[0]
        rows = r * _BLK + jax.lax.broadcasted_iota(jnp.int32, (_BLK, 1), 0)
        emd = jnp.where(rows < sym, tab_ref[...], last_ref[...])
        emd_ref[...] = jnp.where(rows == sym, jnp.float32(0.0), emd)

    out_ref[...] = in_ref[...] + emd_ref[...][None]


def kernel(inputs, sym_index, table):
    batch = inputs.shape[0]
    sym = jnp.asarray(sym_index, jnp.int32).reshape(1)
    last = table[-1:, :]
    grid = (_ROWS // _BLK, batch)
    return pl.pallas_call(
        _body,
        grid_spec=pltpu.PrefetchScalarGridSpec(
            num_scalar_prefetch=1,
            grid=grid,
            in_specs=[
                pl.BlockSpec((1, _BLK, _WIDTH), lambda r, b, sym: (b, r, 0)),
                pl.BlockSpec((_BLK, _WIDTH), lambda r, b, sym: (r, 0)),
                pl.BlockSpec((1, _WIDTH), lambda r, b, sym: (0, 0)),
            ],
            out_specs=pl.BlockSpec((1, _BLK, _WIDTH), lambda r, b, sym: (b, r, 0)),
            scratch_shapes=[pltpu.VMEM((_BLK, _WIDTH), jnp.float32)],
        ),
        out_shape=jax.ShapeDtypeStruct(inputs.shape, inputs.dtype),
    )(sym, inputs, table, last)
